# baseline (speedup 1.0000x reference)
import functools

import jax
import jax.numpy as jnp
from jax import lax
from jax.experimental import pallas as pl
from jax.experimental.pallas import tpu as pltpu
from jax.experimental.pallas import tpu_sc as plsc

_BATCH = 4096
_SEQ = 200
_D = 64
_B = _BATCH * _SEQ
_NW = 32
_PER_W = _B // _NW
_CH = 128
_NCH = _PER_W // _CH


def _emb_call(xw, table):
    mesh = plsc.VectorSubcoreMesh(core_axis_name="c", subcore_axis_name="s")

    @functools.partial(
        pl.kernel,
        mesh=mesh,
        out_type=jax.ShapeDtypeStruct((_B, _D), jnp.float32),
        scratch_types=[
            pltpu.VMEM((_NCH, _CH), jnp.int32),
            pltpu.VMEM((_CH, _D), jnp.float32),
            pltpu.SemaphoreType.DMA,
        ],
        compiler_params=pltpu.CompilerParams(use_tc_tiling_on_sc=False),
    )
    def k(x_hbm, table_hbm, out_hbm, idx_v, rows_v, gsem):
        wid = lax.axis_index("s") * 2 + lax.axis_index("c")
        base = wid * _PER_W
        pltpu.sync_copy(x_hbm.at[wid], idx_v)

        def body(j, carry):
            pltpu.async_copy(table_hbm.at[idx_v.at[j]], rows_v, gsem).wait()
            pltpu.sync_copy(rows_v, out_hbm.at[pl.ds(base + j * _CH, _CH)])
            return carry

        lax.fori_loop(0, _NCH, body, 0)

    return k(xw, table)


def kernel(x, table):
    xw = x.reshape(_NW, _NCH, _CH).astype(jnp.int32)
    out = _emb_call(xw, table)
    return out.reshape(_BATCH, _SEQ, _D)


# speedup vs baseline: 1.1134x; 1.1134x over previous
import functools

import jax
import jax.numpy as jnp
from jax import lax
from jax.experimental import pallas as pl
from jax.experimental.pallas import tpu as pltpu
from jax.experimental.pallas import tpu_sc as plsc

_BATCH = 4096
_SEQ = 200
_D = 64
_B = _BATCH * _SEQ
_NW = 32
_PER_W = _B // _NW
_CH = 128
_NCH = _PER_W // _CH
_GPS = 4
_SLOT = _CH * _GPS
_NST = _PER_W // _SLOT


def _emb_call(xw, table):
    mesh = plsc.VectorSubcoreMesh(core_axis_name="c", subcore_axis_name="s")

    @functools.partial(
        pl.kernel,
        mesh=mesh,
        out_type=jax.ShapeDtypeStruct((_B, _D), jnp.float32),
        scratch_types=[
            pltpu.VMEM((_NCH, _CH), jnp.int32),
            pltpu.VMEM((2, _SLOT, _D), jnp.float32),
            pltpu.SemaphoreType.DMA,
            pltpu.SemaphoreType.DMA,
            pltpu.SemaphoreType.DMA,
            pltpu.SemaphoreType.DMA,
        ],
        compiler_params=pltpu.CompilerParams(use_tc_tiling_on_sc=False),
    )
    def k(x_hbm, table_hbm, out_hbm, idx_v, rows_v, g0, g1, w0, w1):
        wid = lax.axis_index("s") * 2 + lax.axis_index("c")
        base = wid * _PER_W
        pltpu.sync_copy(x_hbm.at[wid], idx_v)

        def body(i, carry):
            for s, gsem, wsem in ((0, g0, w0), (1, g1, w1)):
                g = 2 * i + s
                dst = out_hbm.at[pl.ds(base + g * _SLOT, _SLOT)]
                slot = rows_v.at[s]

                @pl.when(g >= 2)
                def _():
                    pltpu.make_async_copy(slot, dst, wsem).wait()

                descs = []
                for q in range(_GPS):
                    j = g * _GPS + q
                    descs.append(pltpu.async_copy(
                        table_hbm.at[idx_v.at[j]],
                        slot.at[pl.ds(q * _CH, _CH)],
                        gsem,
                    ))
                for d in descs:
                    d.wait()
                pltpu.async_copy(slot, dst, wsem)
            return carry

        lax.fori_loop(0, _NST // 2, body, 0)

        for s, wsem in ((0, w0), (1, w1)):
            g_last = _NST - 2 + s
            pltpu.make_async_copy(
                rows_v.at[s],
                out_hbm.at[pl.ds(base + g_last * _SLOT, _SLOT)],
                wsem,
            ).wait()

    return k(xw, table)


def kernel(x, table):
    xw = x.reshape(_NW, _NCH, _CH).astype(jnp.int32)
    out = _emb_call(xw, table)
    return out.reshape(_BATCH, _SEQ, _D)
